# baseline (speedup 1.0000x reference)
import functools

import jax
import jax.numpy as jnp
from jax import lax
from jax.experimental import pallas as pl
from jax.experimental.pallas import tpu as pltpu
from jax.experimental.pallas import tpu_sc as plsc

N = 10000
E = 320000
D = 128

NS = 16
B = 48
NBT = 420
EPAD = NS * NBT * B
NPAD = 10240
RPT = NPAD // NS
DUMP = NPAD - 1

_mesh = plsc.VectorSubcoreMesh(core_axis_name="c", subcore_axis_name="s")


@functools.partial(
    pl.kernel,
    mesh=_mesh,
    out_type=[
        jax.ShapeDtypeStruct((2, NPAD, D), jnp.float32),
        jax.ShapeDtypeStruct((2, NPAD), jnp.float32),
    ],
    scratch_types=[
        pltpu.VMEM((2, B), jnp.int32),
        pltpu.VMEM((2, B), jnp.int32),
        pltpu.VMEM((2, B), jnp.float32),
        pltpu.VMEM((2, B), jnp.float32),
        pltpu.VMEM((B, 2 * D), jnp.float32),
        pltpu.VMEM((B, 2 * D), jnp.float32),
        pltpu.VMEM((B, D), jnp.float32),
        pltpu.VMEM((B,), jnp.float32),
        pltpu.VMEM_SHARED((NPAD, D), jnp.float32),
        pltpu.VMEM_SHARED((NPAD,), jnp.float32),
        pltpu.SemaphoreType.DMA,
        pltpu.SemaphoreType.DMA,
    ],
)
def _sc_aggregate(pk_hbm, edi_hbm, edw_hbm, z128_hbm, z1_hbm,
                  out_hbm, deg_hbm,
                  edi0, edi1, edw0, edw1, pk0, pk1, msg_b, ones_b,
                  acc, dacc, se, sg):
    c = lax.axis_index("c")
    s = lax.axis_index("s")

    pltpu.sync_copy(z128_hbm, acc.at[pl.ds(s * RPT, RPT)])
    pltpu.sync_copy(z1_hbm, dacc.at[pl.ds(s * RPT, RPT)])

    def fill_ones(g, t):
        ones_b[pl.ds(g * 16, 16)] = jnp.ones((16,), jnp.float32)
        return t
    lax.fori_loop(0, B // 16, fill_ones, 0)

    plsc.subcore_barrier()

    def compute_mean(edw_v, buf):
        def group_body(g, t2):
            w16 = edw_v[0, pl.ds(g * 16, 16)]
            for jj in range(16):
                wm = w16[jj]
                j = g * 16 + jj
                for sl in range(D // 16):
                    msg_b[j, pl.ds(sl * 16, 16)] = buf[j, pl.ds(sl * 16, 16)] * wm
            return t2
        lax.fori_loop(0, B // 16, group_body, 0)

    def compute_var(edw_v, buf):
        def group_body(g, t2):
            w16m = edw_v[0, pl.ds(g * 16, 16)]
            w16v = edw_v[1, pl.ds(g * 16, 16)]
            w16m2 = w16m * w16m
            for jj in range(16):
                wm2 = w16m2[jj]
                wv = w16v[jj]
                j = g * 16 + jj
                for sl in range(D // 16):
                    hm = buf[j, pl.ds(sl * 16, 16)]
                    hv = buf[j, pl.ds(D + sl * 16, 16)]
                    msg_b[j, pl.ds(sl * 16, 16)] = hv * wm2 + hm * hm * wv
            return t2
        lax.fori_loop(0, B // 16, group_body, 0)

    def run_loop(compute_batch, do_deg):
        pltpu.async_copy(edi_hbm.at[s, 0], edi0, se)
        pltpu.async_copy(edw_hbm.at[s, 0], edw0, se)
        pltpu.async_copy(edi_hbm.at[s, 1], edi1, se)
        pltpu.async_copy(edw_hbm.at[s, 1], edw1, se)
        pltpu.make_async_copy(edi_hbm.at[s, 0], edi0, se).wait()
        pltpu.make_async_copy(edw_hbm.at[s, 0], edw0, se).wait()
        pltpu.async_copy(pk_hbm.at[edi0.at[1]], pk0, sg)

        def pair_body(i2, t):
            for k in (0, 1):
                edi_v, edw_v, buf = (edi0, edw0, pk0) if k == 0 else (edi1, edw1, pk1)
                edi_n, edw_n, buf_n = (edi1, edw1, pk1) if k == 0 else (edi0, edw0, pk0)

                def body(i):
                    @pl.when(i + 1 < NBT)
                    def _():
                        pltpu.make_async_copy(edi_hbm.at[s, i + 1], edi_n, se).wait()
                        pltpu.make_async_copy(edw_hbm.at[s, i + 1], edw_n, se).wait()
                        pltpu.async_copy(pk_hbm.at[edi_n.at[1]], buf_n, sg)

                    compute_batch(edw_v, buf)
                    pltpu.make_async_copy(pk_hbm.at[edi_v.at[1]], buf, sg).wait()
                    pltpu.sync_copy(msg_b, acc.at[edi_v.at[0]], add=True)
                    if do_deg:
                        pltpu.sync_copy(ones_b, dacc.at[edi_v.at[0]], add=True)

                    @pl.when(i + 2 < NBT)
                    def _():
                        pltpu.async_copy(edi_hbm.at[s, i + 2], edi_v, se)
                        pltpu.async_copy(edw_hbm.at[s, i + 2], edw_v, se)

                body(i2 * 2 + k)
            return t
        lax.fori_loop(0, NBT // 2, pair_body, 0)

    def mean_loop():
        run_loop(compute_mean, True)

    def var_loop():
        run_loop(compute_var, False)

    pl.when(c == 0)(mean_loop)
    pl.when(c == 1)(var_loop)

    plsc.subcore_barrier()

    pltpu.sync_copy(acc.at[pl.ds(s * RPT, RPT)], out_hbm.at[c, pl.ds(s * RPT, RPT)])
    pltpu.sync_copy(dacc.at[pl.ds(s * RPT, RPT)], deg_hbm.at[c, pl.ds(s * RPT, RPT)])


def _mm_body(x_ref, wm_ref, bm_ref, wl_ref, bl_ref, pk_ref):
    x = x_ref[...]
    pk_ref[:, :D] = jnp.dot(x, wm_ref[...], preferred_element_type=jnp.float32) + bm_ref[...]
    pk_ref[:, D:] = jnp.exp(
        jnp.dot(x, wl_ref[...], preferred_element_type=jnp.float32) + bl_ref[...])


def _fin_body(mr_ref, vr_ref, d_ref, g_ref, b_ref, om_ref, ov_ref):
    d = jnp.maximum(d_ref[...], 1.0)
    m = mr_ref[...] / d
    ov_ref[...] = vr_ref[...] / (d * d)
    mu = jnp.mean(m, axis=1, keepdims=True)
    var = jnp.mean((m - mu) ** 2, axis=1, keepdims=True)
    om_ref[...] = (m - mu) * lax.rsqrt(var + 1e-5) * g_ref[...] + b_ref[...]


_MM_ROWS = 1000


def kernel(x, edge_index, edge_weight_mean, edge_weight_var,
           W_mean, b_mean, W_logvar, b_logvar, ln_gamma, ln_beta):
    [pk] = pl.pallas_call(
        _mm_body,
        grid=(N // _MM_ROWS,),
        in_specs=[
            pl.BlockSpec((_MM_ROWS, D), lambda i: (i, 0)),
            pl.BlockSpec((D, D), lambda i: (0, 0)),
            pl.BlockSpec((D,), lambda i: (0,)),
            pl.BlockSpec((D, D), lambda i: (0, 0)),
            pl.BlockSpec((D,), lambda i: (0,)),
        ],
        out_specs=[pl.BlockSpec((_MM_ROWS, 2 * D), lambda i: (i, 0))],
        out_shape=[jax.ShapeDtypeStruct((N, 2 * D), jnp.float32)],
    )(x, W_mean, b_mean, W_logvar, b_logvar)

    npad_e = EPAD - E
    row_p = jnp.concatenate([edge_index[0], jnp.full((npad_e,), DUMP, jnp.int32)])
    col_p = jnp.concatenate([edge_index[1], jnp.zeros((npad_e,), jnp.int32)])
    ewm_p = jnp.concatenate([edge_weight_mean, jnp.zeros((npad_e,), jnp.float32)])
    ewv_p = jnp.concatenate([edge_weight_var, jnp.zeros((npad_e,), jnp.float32)])
    edi = jnp.stack([row_p, col_p], axis=0).reshape(2, NS, NBT, B).transpose(1, 2, 0, 3)
    edw = jnp.stack([ewm_p, ewv_p], axis=0).reshape(2, NS, NBT, B).transpose(1, 2, 0, 3)

    z128 = jnp.zeros((RPT, D), jnp.float32)
    z1 = jnp.zeros((RPT,), jnp.float32)
    out_raw, deg_raw = _sc_aggregate(pk, edi, edw, z128, z1)

    mean_raw = out_raw[0, :N]
    var_raw = out_raw[1, :N]
    deg = deg_raw[0, :N, None]

    out_mean_ln, out_var = pl.pallas_call(
        _fin_body,
        grid=(N // _MM_ROWS,),
        in_specs=[
            pl.BlockSpec((_MM_ROWS, D), lambda i: (i, 0)),
            pl.BlockSpec((_MM_ROWS, D), lambda i: (i, 0)),
            pl.BlockSpec((_MM_ROWS, 1), lambda i: (i, 0)),
            pl.BlockSpec((D,), lambda i: (0,)),
            pl.BlockSpec((D,), lambda i: (0,)),
        ],
        out_specs=[pl.BlockSpec((_MM_ROWS, D), lambda i: (i, 0))] * 2,
        out_shape=[jax.ShapeDtypeStruct((N, D), jnp.float32)] * 2,
    )(mean_raw, var_raw, deg, ln_gamma, ln_beta)

    return (out_mean_ln, out_var)


# speedup vs baseline: 2.0911x; 2.0911x over previous
import functools

import jax
import jax.numpy as jnp
from jax import lax
from jax.experimental import pallas as pl
from jax.experimental.pallas import tpu as pltpu
from jax.experimental.pallas import tpu_sc as plsc

N = 10000
E = 320000
D = 128

NS = 16
B = 96
NBT = 210
EPAD = NS * NBT * B
NPAD = 10240
RPT = NPAD // NS
DUMP = NPAD - 1

_mesh = plsc.VectorSubcoreMesh(core_axis_name="c", subcore_axis_name="s")


@functools.partial(
    pl.kernel,
    mesh=_mesh,
    out_type=[
        jax.ShapeDtypeStruct((2, NPAD, D), jnp.float32),
        jax.ShapeDtypeStruct((2, NPAD), jnp.float32),
    ],
    scratch_types=[
        pltpu.VMEM((2, B), jnp.int32),
        pltpu.VMEM((2, B), jnp.int32),
        pltpu.VMEM((2, B), jnp.float32),
        pltpu.VMEM((2, B), jnp.float32),
        pltpu.VMEM((B, D), jnp.float32),
        pltpu.VMEM((B, D), jnp.float32),
        pltpu.VMEM((B, D), jnp.float32),
        pltpu.VMEM((B,), jnp.float32),
        pltpu.VMEM_SHARED((NPAD, D), jnp.float32),
        pltpu.VMEM_SHARED((NPAD,), jnp.float32),
        pltpu.SemaphoreType.DMA,
        pltpu.SemaphoreType.DMA,
    ],
)
def _sc_aggregate(pka_hbm, pkb_hbm, edi_hbm, edw_hbm, z128_hbm, z1_hbm,
                  out_hbm, deg_hbm,
                  edi0, edi1, edw0, edw1, pk0, pk1, msg_b, ones_b,
                  acc, dacc, se, sg):
    c = lax.axis_index("c")
    s = lax.axis_index("s")

    pltpu.sync_copy(z128_hbm, acc.at[pl.ds(s * RPT, RPT)])
    pltpu.sync_copy(z1_hbm, dacc.at[pl.ds(s * RPT, RPT)])

    def fill_ones(g, t):
        ones_b[pl.ds(g * 16, 16)] = jnp.ones((16,), jnp.float32)
        return t
    lax.fori_loop(0, B // 16, fill_ones, 0)

    plsc.subcore_barrier()

    def compute_both(edw_v, buf):
        def group_body(g, t2):
            w16m = edw_v[0, pl.ds(g * 16, 16)]
            w16v = edw_v[1, pl.ds(g * 16, 16)]
            w16m2 = w16m * w16m
            for jj in range(16):
                wm = w16m[jj]
                wm2 = w16m2[jj]
                wv = w16v[jj]
                j = g * 16 + jj
                for sl in range(D // 32):
                    hm = buf[j, pl.ds(sl * 16, 16)]
                    hv = buf[j, pl.ds(D // 2 + sl * 16, 16)]
                    msg_b[j, pl.ds(sl * 16, 16)] = hm * wm
                    msg_b[j, pl.ds(D // 2 + sl * 16, 16)] = hv * wm2 + hm * hm * wv
            return t2
        lax.fori_loop(0, B // 16, group_body, 0)

    def run_loop(pk_hbm, do_deg):
        pltpu.async_copy(edi_hbm.at[s, 0], edi0, se)
        pltpu.async_copy(edw_hbm.at[s, 0], edw0, se)
        pltpu.async_copy(edi_hbm.at[s, 1], edi1, se)
        pltpu.async_copy(edw_hbm.at[s, 1], edw1, se)
        pltpu.make_async_copy(edi_hbm.at[s, 0], edi0, se).wait()
        pltpu.make_async_copy(edw_hbm.at[s, 0], edw0, se).wait()
        pltpu.async_copy(pk_hbm.at[edi0.at[1]], pk0, sg)

        def pair_body(i2, t):
            for k in (0, 1):
                edi_v, edw_v, buf = (edi0, edw0, pk0) if k == 0 else (edi1, edw1, pk1)
                edi_n, edw_n, buf_n = (edi1, edw1, pk1) if k == 0 else (edi0, edw0, pk0)

                def body(i):
                    @pl.when(i + 1 < NBT)
                    def _():
                        pltpu.make_async_copy(edi_hbm.at[s, i + 1], edi_n, se).wait()
                        pltpu.make_async_copy(edw_hbm.at[s, i + 1], edw_n, se).wait()
                        pltpu.async_copy(pk_hbm.at[edi_n.at[1]], buf_n, sg)

                    pltpu.make_async_copy(pk_hbm.at[edi_v.at[1]], buf, sg).wait()
                    compute_both(edw_v, buf)
                    pltpu.sync_copy(msg_b, acc.at[edi_v.at[0]], add=True)
                    if do_deg:
                        pltpu.sync_copy(ones_b, dacc.at[edi_v.at[0]], add=True)

                    @pl.when(i + 2 < NBT)
                    def _():
                        pltpu.async_copy(edi_hbm.at[s, i + 2], edi_v, se)
                        pltpu.async_copy(edw_hbm.at[s, i + 2], edw_v, se)

                body(i2 * 2 + k)
            return t
        lax.fori_loop(0, NBT // 2, pair_body, 0)

    def mean_loop():
        run_loop(pka_hbm, True)

    def var_loop():
        run_loop(pkb_hbm, False)

    pl.when(c == 0)(mean_loop)
    pl.when(c == 1)(var_loop)

    plsc.subcore_barrier()

    pltpu.sync_copy(acc.at[pl.ds(s * RPT, RPT)], out_hbm.at[c, pl.ds(s * RPT, RPT)])
    pltpu.sync_copy(dacc.at[pl.ds(s * RPT, RPT)], deg_hbm.at[c, pl.ds(s * RPT, RPT)])


def _mm_body(x_ref, wm_ref, bm_ref, wl_ref, bl_ref, pka_ref, pkb_ref):
    x = x_ref[...]
    hm = jnp.dot(x, wm_ref[...], preferred_element_type=jnp.float32) + bm_ref[...]
    hv = jnp.exp(jnp.dot(x, wl_ref[...], preferred_element_type=jnp.float32) + bl_ref[...])
    h = D // 2
    pka_ref[:, :h] = hm[:, :h]
    pka_ref[:, h:] = hv[:, :h]
    pkb_ref[:, :h] = hm[:, h:]
    pkb_ref[:, h:] = hv[:, h:]


def _fin_body(mr_ref, vr_ref, d_ref, g_ref, b_ref, om_ref, ov_ref):
    d = jnp.maximum(d_ref[...], 1.0)
    m = mr_ref[...] / d
    ov_ref[...] = vr_ref[...] / (d * d)
    mu = jnp.mean(m, axis=1, keepdims=True)
    var = jnp.mean((m - mu) ** 2, axis=1, keepdims=True)
    om_ref[...] = (m - mu) * lax.rsqrt(var + 1e-5) * g_ref[...] + b_ref[...]


_MM_ROWS = 1000


def kernel(x, edge_index, edge_weight_mean, edge_weight_var,
           W_mean, b_mean, W_logvar, b_logvar, ln_gamma, ln_beta):
    pka, pkb = pl.pallas_call(
        _mm_body,
        grid=(N // _MM_ROWS,),
        in_specs=[
            pl.BlockSpec((_MM_ROWS, D), lambda i: (i, 0)),
            pl.BlockSpec((D, D), lambda i: (0, 0)),
            pl.BlockSpec((D,), lambda i: (0,)),
            pl.BlockSpec((D, D), lambda i: (0, 0)),
            pl.BlockSpec((D,), lambda i: (0,)),
        ],
        out_specs=[pl.BlockSpec((_MM_ROWS, D), lambda i: (i, 0))] * 2,
        out_shape=[jax.ShapeDtypeStruct((N, D), jnp.float32)] * 2,
    )(x, W_mean, b_mean, W_logvar, b_logvar)

    npad_e = EPAD - E
    row_p = jnp.concatenate([edge_index[0], jnp.full((npad_e,), DUMP, jnp.int32)])
    col_p = jnp.concatenate([edge_index[1], jnp.zeros((npad_e,), jnp.int32)])
    ewm_p = jnp.concatenate([edge_weight_mean, jnp.zeros((npad_e,), jnp.float32)])
    ewv_p = jnp.concatenate([edge_weight_var, jnp.zeros((npad_e,), jnp.float32)])
    edi = jnp.stack([row_p, col_p], axis=0).reshape(2, NS, NBT, B).transpose(1, 2, 0, 3)
    edw = jnp.stack([ewm_p, ewv_p], axis=0).reshape(2, NS, NBT, B).transpose(1, 2, 0, 3)

    z128 = jnp.zeros((RPT, D), jnp.float32)
    z1 = jnp.zeros((RPT,), jnp.float32)
    out_raw, deg_raw = _sc_aggregate(pka, pkb, edi, edw, z128, z1)

    h = D // 2
    mean_raw = jnp.concatenate([out_raw[0, :N, :h], out_raw[1, :N, :h]], axis=1)
    var_raw = jnp.concatenate([out_raw[0, :N, h:], out_raw[1, :N, h:]], axis=1)
    deg = deg_raw[0, :N, None]

    out_mean_ln, out_var = pl.pallas_call(
        _fin_body,
        grid=(N // _MM_ROWS,),
        in_specs=[
            pl.BlockSpec((_MM_ROWS, D), lambda i: (i, 0)),
            pl.BlockSpec((_MM_ROWS, D), lambda i: (i, 0)),
            pl.BlockSpec((_MM_ROWS, 1), lambda i: (i, 0)),
            pl.BlockSpec((D,), lambda i: (0,)),
            pl.BlockSpec((D,), lambda i: (0,)),
        ],
        out_specs=[pl.BlockSpec((_MM_ROWS, D), lambda i: (i, 0))] * 2,
        out_shape=[jax.ShapeDtypeStruct((N, D), jnp.float32)] * 2,
    )(mean_raw, var_raw, deg, ln_gamma, ln_beta)

    return (out_mean_ln, out_var)


# speedup vs baseline: 2.1647x; 1.0352x over previous
import functools

import jax
import jax.numpy as jnp
from jax import lax
from jax.experimental import pallas as pl
from jax.experimental.pallas import tpu as pltpu
from jax.experimental.pallas import tpu_sc as plsc

N = 10000
E = 320000
D = 128

NS = 16
B = 96
NBT = 210
EPAD = NS * NBT * B
NPAD = 10240
RPT = NPAD // NS
DUMP = NPAD - 1

_mesh = plsc.VectorSubcoreMesh(core_axis_name="c", subcore_axis_name="s")


@functools.partial(
    pl.kernel,
    mesh=_mesh,
    out_type=[
        jax.ShapeDtypeStruct((2, NPAD, D), jnp.float32),
        jax.ShapeDtypeStruct((2, NPAD), jnp.float32),
    ],
    scratch_types=[
        pltpu.VMEM((2, B), jnp.int32),
        pltpu.VMEM((2, B), jnp.int32),
        pltpu.VMEM((2, B), jnp.float32),
        pltpu.VMEM((2, B), jnp.float32),
        pltpu.VMEM((B, D), jnp.float32),
        pltpu.VMEM((B, D), jnp.float32),
        pltpu.VMEM((B, D), jnp.float32),
        pltpu.VMEM((B,), jnp.float32),
        pltpu.VMEM_SHARED((NPAD, D), jnp.float32),
        pltpu.VMEM_SHARED((NPAD,), jnp.float32),
        pltpu.SemaphoreType.DMA,
        pltpu.SemaphoreType.DMA,
    ],
)
def _sc_aggregate(pka_hbm, pkb_hbm, edi_hbm, edw_hbm, z128_hbm, z1_hbm,
                  out_hbm, deg_hbm,
                  edi0, edi1, edw0, edw1, pk0, pk1, msg_b, ones_b,
                  acc, dacc, se, sg):
    c = lax.axis_index("c")
    s = lax.axis_index("s")

    pltpu.sync_copy(z128_hbm, acc.at[pl.ds(s * RPT, RPT)])
    pltpu.sync_copy(z1_hbm, dacc.at[pl.ds(s * RPT, RPT)])

    def fill_ones(g, t):
        ones_b[pl.ds(g * 16, 16)] = jnp.ones((16,), jnp.float32)
        return t
    lax.fori_loop(0, B // 16, fill_ones, 0)

    plsc.subcore_barrier()

    def compute_both(edw_v, buf):
        def group_body(g, t2):
            w16m = edw_v[0, pl.ds(g * 16, 16)]
            w16v = edw_v[1, pl.ds(g * 16, 16)]
            w16m2 = w16m * w16m
            for jj in range(16):
                wm = w16m[jj]
                wm2 = w16m2[jj]
                wv = w16v[jj]
                j = g * 16 + jj
                for sl in range(D // 32):
                    hm = buf[j, pl.ds(sl * 16, 16)]
                    hv = buf[j, pl.ds(D // 2 + sl * 16, 16)]
                    msg_b[j, pl.ds(sl * 16, 16)] = hm * wm
                    msg_b[j, pl.ds(D // 2 + sl * 16, 16)] = hv * wm2 + hm * hm * wv
            return t2
        lax.fori_loop(0, B // 16, group_body, 0)

    def run_loop(pk_hbm, do_deg):
        pltpu.async_copy(edi_hbm.at[s, 0], edi0, se)
        pltpu.async_copy(edw_hbm.at[s, 0], edw0, se)
        pltpu.async_copy(edi_hbm.at[s, 1], edi1, se)
        pltpu.async_copy(edw_hbm.at[s, 1], edw1, se)
        pltpu.make_async_copy(edi_hbm.at[s, 0], edi0, se).wait()
        pltpu.make_async_copy(edw_hbm.at[s, 0], edw0, se).wait()
        pltpu.async_copy(pk_hbm.at[edi0.at[1]], pk0, sg)

        def pair_body(i2, t):
            for k in (0, 1):
                edi_v, edw_v, buf = (edi0, edw0, pk0) if k == 0 else (edi1, edw1, pk1)
                edi_n, edw_n, buf_n = (edi1, edw1, pk1) if k == 0 else (edi0, edw0, pk0)

                def body(i):
                    @pl.when(i + 1 < NBT)
                    def _():
                        pltpu.make_async_copy(edi_hbm.at[s, i + 1], edi_n, se).wait()
                        pltpu.make_async_copy(edw_hbm.at[s, i + 1], edw_n, se).wait()
                        pltpu.async_copy(pk_hbm.at[edi_n.at[1]], buf_n, sg)

                    pltpu.make_async_copy(pk_hbm.at[edi_v.at[1]], buf, sg).wait()
                    compute_both(edw_v, buf)
                    pltpu.sync_copy(msg_b, acc.at[edi_v.at[0]], add=True)
                    if do_deg:
                        pltpu.sync_copy(ones_b, dacc.at[edi_v.at[0]], add=True)

                    @pl.when(i + 2 < NBT)
                    def _():
                        pltpu.async_copy(edi_hbm.at[s, i + 2], edi_v, se)
                        pltpu.async_copy(edw_hbm.at[s, i + 2], edw_v, se)

                body(i2 * 2 + k)
            return t
        lax.fori_loop(0, NBT // 2, pair_body, 0)

    def mean_loop():
        run_loop(pka_hbm, True)

    def var_loop():
        run_loop(pkb_hbm, False)

    pl.when(c == 0)(mean_loop)
    pl.when(c == 1)(var_loop)

    plsc.subcore_barrier()

    pltpu.sync_copy(acc.at[pl.ds(s * RPT, RPT)], out_hbm.at[c, pl.ds(s * RPT, RPT)])
    pltpu.sync_copy(dacc.at[pl.ds(s * RPT, RPT)], deg_hbm.at[c, pl.ds(s * RPT, RPT)])


def _mm_body(x_ref, wm_ref, bm_ref, wl_ref, bl_ref, pka_ref, pkb_ref):
    x = x_ref[...]
    hm = jnp.dot(x, wm_ref[...], preferred_element_type=jnp.float32) + bm_ref[...]
    hv = jnp.exp(jnp.dot(x, wl_ref[...], preferred_element_type=jnp.float32) + bl_ref[...])
    h = D // 2
    pka_ref[:, :h] = hm[:, :h]
    pka_ref[:, h:] = hv[:, :h]
    pkb_ref[:, :h] = hm[:, h:]
    pkb_ref[:, h:] = hv[:, h:]


def _fin_body(a_ref, b2_ref, d_ref, g_ref, b_ref, om_ref, ov_ref):
    h = D // 2
    a = a_ref[0]
    bb = b2_ref[0]
    mr = jnp.concatenate([a[:, :h], bb[:, :h]], axis=1)
    vr = jnp.concatenate([a[:, h:], bb[:, h:]], axis=1)
    d = jnp.maximum(d_ref[...], 1.0)
    m = mr / d
    ov_ref[...] = vr / (d * d)
    mu = jnp.mean(m, axis=1, keepdims=True)
    var = jnp.mean((m - mu) ** 2, axis=1, keepdims=True)
    om_ref[...] = (m - mu) * lax.rsqrt(var + 1e-5) * g_ref[...] + b_ref[...]


_MM_ROWS = 1000


def kernel(x, edge_index, edge_weight_mean, edge_weight_var,
           W_mean, b_mean, W_logvar, b_logvar, ln_gamma, ln_beta):
    pka, pkb = pl.pallas_call(
        _mm_body,
        grid=(N // _MM_ROWS,),
        in_specs=[
            pl.BlockSpec((_MM_ROWS, D), lambda i: (i, 0)),
            pl.BlockSpec((D, D), lambda i: (0, 0)),
            pl.BlockSpec((D,), lambda i: (0,)),
            pl.BlockSpec((D, D), lambda i: (0, 0)),
            pl.BlockSpec((D,), lambda i: (0,)),
        ],
        out_specs=[pl.BlockSpec((_MM_ROWS, D), lambda i: (i, 0))] * 2,
        out_shape=[jax.ShapeDtypeStruct((N, D), jnp.float32)] * 2,
    )(x, W_mean, b_mean, W_logvar, b_logvar)

    npad_e = EPAD - E
    row_p = jnp.concatenate([edge_index[0], jnp.full((npad_e,), DUMP, jnp.int32)])
    col_p = jnp.concatenate([edge_index[1], jnp.zeros((npad_e,), jnp.int32)])
    ewm_p = jnp.concatenate([edge_weight_mean, jnp.zeros((npad_e,), jnp.float32)])
    ewv_p = jnp.concatenate([edge_weight_var, jnp.zeros((npad_e,), jnp.float32)])
    edi = jnp.stack([row_p, col_p], axis=0).reshape(2, NS, NBT, B).transpose(1, 2, 0, 3)
    edw = jnp.stack([ewm_p, ewv_p], axis=0).reshape(2, NS, NBT, B).transpose(1, 2, 0, 3)

    z128 = jnp.zeros((RPT, D), jnp.float32)
    z1 = jnp.zeros((RPT,), jnp.float32)
    out_raw, deg_raw = _sc_aggregate(pka, pkb, edi, edw, z128, z1)

    h = D // 2
    mean_raw = jnp.concatenate([out_raw[0, :N, :h], out_raw[1, :N, :h]], axis=1)
    var_raw = jnp.concatenate([out_raw[0, :N, h:], out_raw[1, :N, h:]], axis=1)
    deg = deg_raw[0, :N, None]

    out_mean_ln, out_var = pl.pallas_call(
        _fin_body,
        grid=(N // _MM_ROWS,),
        in_specs=[
            pl.BlockSpec((1, _MM_ROWS, D), lambda i: (0, i, 0)),
            pl.BlockSpec((1, _MM_ROWS, D), lambda i: (1, i, 0)),
            pl.BlockSpec((_MM_ROWS, 1), lambda i: (i, 0)),
            pl.BlockSpec((D,), lambda i: (0,)),
            pl.BlockSpec((D,), lambda i: (0,)),
        ],
        out_specs=[pl.BlockSpec((_MM_ROWS, D), lambda i: (i, 0))] * 2,
        out_shape=[jax.ShapeDtypeStruct((N, D), jnp.float32)] * 2,
    )(out_raw, out_raw, deg, ln_gamma, ln_beta)

    return (out_mean_ln, out_var)
